# baseline (speedup 1.0000x reference)
import functools

import jax
import jax.numpy as jnp
from jax import lax
from jax.experimental import pallas as pl
from jax.experimental.pallas import tpu as pltpu
from jax.experimental.pallas import tpu_sc as plsc

_LANES = 16
_NC, _NS = 2, 16
_NW = _NC * _NS


def _hist_body(L, V, P, rows_per_w, idx_hbm, counts_hbm, idx_v, c_a, c_b,
               sem_a, sem_b):
    wid = lax.axis_index("s") * _NC + lax.axis_index("c")
    row0 = wid * rows_per_w
    pltpu.sync_copy(idx_hbm.at[pl.ds(row0, rows_per_w), :], idx_v)

    n_full = L // _LANES
    rem = L % _LANES
    lane = lax.iota(jnp.int32, _LANES)
    ones = jnp.ones((_LANES,), jnp.float32)
    zeros = jnp.zeros((_LANES,), jnp.float32)
    full_p = lane * V
    tail_p = jnp.mod(lane + L, _LANES) * V
    tail_mask = lane >= (_LANES - rem)

    def cidx_list(r):
        out = []
        for g in range(n_full):
            out.append(idx_v[r, pl.ds(g * _LANES, _LANES)] + full_p)
        if rem:
            out.append(idx_v[r, pl.ds(L - _LANES, _LANES)] + tail_p)
        return out

    def accumulate(r, buf):
        cs = cidx_list(r)
        for c in cs[:n_full]:
            plsc.addupdate_scatter(buf, [c], ones)
        if rem:
            plsc.addupdate_scatter(buf, [cs[-1]], ones, mask=tail_mask)

    def rezero(r, buf):
        cs = cidx_list(r)
        for c in cs[:n_full]:
            plsc.store_scatter(buf, [c], zeros)
        if rem:
            plsc.store_scatter(buf, [cs[-1]], zeros, mask=tail_mask)

    def zero_all(buf):
        def zb(i, _):
            buf[pl.ds(i * _LANES, _LANES)] = zeros
            return 0
        lax.fori_loop(0, (P * V) // _LANES, zb, 0)

    bufs = (c_a, c_b)
    sems = (sem_a, sem_b)
    nb = len(bufs)
    for b in bufs:
        zero_all(b)

    for k in range(nb):
        accumulate(k, bufs[k])
        pltpu.async_copy(bufs[k], counts_hbm.at[row0 + k], sems[k])

    def ring_body(i, _):
        r = nb * i
        for k in range(nb):
            buf, sem = bufs[k], sems[k]
            pltpu.make_async_copy(
                buf, counts_hbm.at[row0 + r + k - nb], sem).wait()
            rezero(r + k - nb, buf)
            accumulate(r + k, buf)
            pltpu.async_copy(buf, counts_hbm.at[row0 + r + k], sem)
        return 0

    lax.fori_loop(1, rows_per_w // nb, ring_body, 0)
    for k in range(nb):
        pltpu.make_async_copy(
            bufs[k], counts_hbm.at[row0 + rows_per_w - nb + k], sems[k]).wait()


def _mm_body(n_pos, v, counts_ref, e_ref, p_ref, out_ref, t_hi):
    @pl.when(pl.program_id(0) == 0)
    def _build_t():
        e = e_ref[...]
        for p in range(n_pos):
            t = e * p_ref[p, :][None, :]
            t_hi[p * v:(p + 1) * v, :] = t.astype(jnp.bfloat16)

    cb = counts_ref[...].astype(jnp.bfloat16)
    acc = jnp.dot(cb, t_hi[...], preferred_element_type=jnp.float32)
    inv = lax.rsqrt(jnp.sum(acc * acc, axis=1, keepdims=True))
    out_ref[...] = acc * inv


def kernel(indices, embeddings, pos_encodings):
    B, L = indices.shape
    V, D = embeddings.shape
    P = pos_encodings.shape[0]
    assert B % _NW == 0 and L >= _LANES

    mesh = plsc.VectorSubcoreMesh(core_axis_name="c", subcore_axis_name="s")
    n_split = 2
    bh = B // n_split
    rows_per_w = bh // _NW
    hist = pl.kernel(
        functools.partial(_hist_body, L, V, P, rows_per_w),
        out_type=jax.ShapeDtypeStruct((bh, P * V), jnp.float32),
        mesh=mesh,
        compiler_params=pltpu.CompilerParams(needs_layout_passes=False),
        scratch_types=[
            pltpu.VMEM((rows_per_w, L), jnp.int32),
            pltpu.VMEM((P * V,), jnp.float32),
            pltpu.VMEM((P * V,), jnp.float32),
            pltpu.SemaphoreType.DMA,
            pltpu.SemaphoreType.DMA,
        ],
    )

    bt = 256
    mm = pl.pallas_call(
        functools.partial(_mm_body, P, V),
        grid=(bh // bt,),
        in_specs=[
            pl.BlockSpec((bt, P * V), lambda i: (i, 0)),
            pl.BlockSpec((V, D), lambda i: (0, 0)),
            pl.BlockSpec((P, D), lambda i: (0, 0)),
        ],
        out_specs=pl.BlockSpec((bt, D), lambda i: (i, 0)),
        out_shape=jax.ShapeDtypeStruct((bh, D), jnp.float32),
        scratch_shapes=[
            pltpu.VMEM((P * V, D), jnp.bfloat16),
        ],
    )

    counts = [hist(indices[i * bh:(i + 1) * bh]) for i in range(n_split)]
    outs = [mm(c, embeddings, pos_encodings) for c in counts]
    return jnp.concatenate(outs, axis=0)


# speedup vs baseline: 1.1848x; 1.1848x over previous
import functools

import jax
import jax.numpy as jnp
from jax import lax
from jax.experimental import pallas as pl
from jax.experimental.pallas import tpu as pltpu
from jax.experimental.pallas import tpu_sc as plsc

_LANES = 16
_NC, _NS = 2, 16
_NW = _NC * _NS


def _hist_body(L, V, P, rows_per_w, idx_hbm, counts_hbm, idx_v, c_a, c_b,
               sem_a, sem_b):
    wid = lax.axis_index("s") * _NC + lax.axis_index("c")
    row0 = wid * rows_per_w
    pltpu.sync_copy(idx_hbm.at[pl.ds(row0, rows_per_w), :], idx_v)

    n_full = L // _LANES
    rem = L % _LANES
    lane = lax.iota(jnp.int32, _LANES)
    ones = jnp.ones((_LANES,), jnp.float32)
    zeros = jnp.zeros((_LANES,), jnp.float32)
    full_p = lane * V
    tail_p = jnp.mod(lane + L, _LANES) * V
    tail_mask = lane >= (_LANES - rem)

    def cidx_list(r):
        out = []
        for g in range(n_full):
            out.append(idx_v[r, pl.ds(g * _LANES, _LANES)] + full_p)
        if rem:
            out.append(idx_v[r, pl.ds(L - _LANES, _LANES)] + tail_p)
        return out

    def accumulate(r, buf):
        cs = cidx_list(r)
        for c in cs[:n_full]:
            plsc.addupdate_scatter(buf, [c], ones)
        if rem:
            plsc.addupdate_scatter(buf, [cs[-1]], ones, mask=tail_mask)

    def rezero(r, buf):
        cs = cidx_list(r)
        for c in cs[:n_full]:
            plsc.store_scatter(buf, [c], zeros)
        if rem:
            plsc.store_scatter(buf, [cs[-1]], zeros, mask=tail_mask)

    def zero_all(buf):
        def zb(i, _):
            buf[pl.ds(i * _LANES, _LANES)] = zeros
            return 0
        lax.fori_loop(0, (P * V) // _LANES, zb, 0)

    bufs = (c_a, c_b)
    sems = (sem_a, sem_b)
    nb = len(bufs)
    for b in bufs:
        zero_all(b)

    for k in range(nb):
        accumulate(k, bufs[k])
        pltpu.async_copy(bufs[k], counts_hbm.at[row0 + k], sems[k])

    def ring_body(i, _):
        r = nb * i
        for k in range(nb):
            buf, sem = bufs[k], sems[k]
            pltpu.make_async_copy(
                buf, counts_hbm.at[row0 + r + k - nb], sem).wait()
            rezero(r + k - nb, buf)
            accumulate(r + k, buf)
            pltpu.async_copy(buf, counts_hbm.at[row0 + r + k], sem)
        return 0

    lax.fori_loop(1, rows_per_w // nb, ring_body, 0)
    for k in range(nb):
        pltpu.make_async_copy(
            bufs[k], counts_hbm.at[row0 + rows_per_w - nb + k], sems[k]).wait()


def _mm_body(n_pos, v, counts_ref, e_ref, p_ref, out_ref, t_hi):
    @pl.when(pl.program_id(0) == 0)
    def _build_t():
        e = e_ref[...]
        for p in range(n_pos):
            t = e * p_ref[p, :][None, :]
            t_hi[p * v:(p + 1) * v, :] = t.astype(jnp.bfloat16)

    cb = counts_ref[...].astype(jnp.bfloat16)
    acc = jnp.dot(cb, t_hi[...], preferred_element_type=jnp.float32)
    inv = lax.rsqrt(jnp.sum(acc * acc, axis=1, keepdims=True))
    out_ref[...] = acc * inv


def kernel(indices, embeddings, pos_encodings):
    B, L = indices.shape
    V, D = embeddings.shape
    P = pos_encodings.shape[0]
    rows_per_w = B // _NW
    assert B % _NW == 0 and L >= _LANES

    mesh = plsc.VectorSubcoreMesh(core_axis_name="c", subcore_axis_name="s")
    hist = pl.kernel(
        functools.partial(_hist_body, L, V, P, rows_per_w),
        out_type=jax.ShapeDtypeStruct((B, P * V), jnp.float32),
        mesh=mesh,
        compiler_params=pltpu.CompilerParams(needs_layout_passes=False),
        scratch_types=[
            pltpu.VMEM((rows_per_w, L), jnp.int32),
            pltpu.VMEM((P * V,), jnp.float32),
            pltpu.VMEM((P * V,), jnp.float32),
            pltpu.SemaphoreType.DMA,
            pltpu.SemaphoreType.DMA,
        ],
    )
    counts = hist(indices)

    bt = 512
    out = pl.pallas_call(
        functools.partial(_mm_body, P, V),
        grid=(B // bt,),
        in_specs=[
            pl.BlockSpec((bt, P * V), lambda i: (i, 0)),
            pl.BlockSpec((V, D), lambda i: (0, 0)),
            pl.BlockSpec((P, D), lambda i: (0, 0)),
        ],
        out_specs=pl.BlockSpec((bt, D), lambda i: (i, 0)),
        out_shape=jax.ShapeDtypeStruct((B, D), jnp.float32),
        scratch_shapes=[
            pltpu.VMEM((P * V, D), jnp.bfloat16),
        ],
    )(counts, embeddings, pos_encodings)
    return out
